# baseline (speedup 1.0000x reference)
import functools

import jax
import jax.numpy as jnp
from jax import lax
from jax.experimental import pallas as pl
from jax.experimental.pallas import tpu as pltpu
from jax.experimental.pallas import tpu_sc as plsc

NC = 2
NS = 16
NW = NC * NS
LANES = 16
CHUNK = 128
SC_FRAC_NUM = 35
SC_ALIGN = NW * CHUNK * 2
BLK = 1024


def _make_sc_lookup(n_rows, n_tab, d, dtype):
    per_w = n_rows // NW
    n_chunk = per_w // CHUNK
    tab_words = n_tab * d
    mesh = plsc.VectorSubcoreMesh(core_axis_name="c", subcore_axis_name="s")

    @functools.partial(
        pl.kernel,
        out_type=jax.ShapeDtypeStruct((n_rows * d,), dtype),
        mesh=mesh,
        scratch_types=[
            pltpu.VMEM((tab_words,), dtype),
            pltpu.VMEM((per_w,), jnp.int32),
            pltpu.VMEM((CHUNK * d,), dtype),
            pltpu.VMEM((CHUNK * d,), dtype),
            pltpu.SemaphoreType.DMA,
            pltpu.SemaphoreType.DMA,
        ],
    )
    def lookup(z_hbm, tabf_hbm, out_hbm, tab_v, idx_v, rows0, rows1, p0, p1):
        wid = lax.axis_index("s") * NC + lax.axis_index("c")
        base = wid * per_w

        pltpu.sync_copy(tabf_hbm, tab_v)
        pltpu.sync_copy(z_hbm.at[pl.ds(base, per_w)], idx_v)

        @plsc.parallel_loop(0, per_w, step=LANES)
        def fix(i):
            sl = pl.ds(i, LANES)
            v = idx_v[sl]
            idx_v[sl] = jnp.where(v > -1, (v + 1) * d, (n_tab - 1) * d)

        def do_chunk(j, buf):
            cb = j * CHUNK

            @plsc.parallel_loop(0, CHUNK, step=LANES)
            def group(gb):
                zvec = idx_v[pl.ds(cb + gb, LANES)]
                gbd = gb * d
                for l in range(LANES):
                    off = zvec[l]
                    o = gbd + l * d
                    for jj in range(d // LANES):
                        buf[pl.ds(o + jj * LANES, LANES)] = (
                            tab_v[pl.ds(off + jj * LANES, LANES)])

        def put(j, buf, sem):
            pltpu.async_copy(
                buf,
                out_hbm.at[pl.ds((base + j * CHUNK) * d, CHUNK * d)],
                sem)

        def wait_put(buf, sem):
            pltpu.make_async_copy(
                buf, out_hbm.at[pl.ds(base * d, CHUNK * d)], sem).wait()

        def body(cc, carry):
            for b, (buf, sem) in enumerate(((rows0, p0), (rows1, p1))):
                @pl.when(cc > 0)
                def _():
                    wait_put(buf, sem)

                do_chunk(2 * cc + b, buf)
                put(2 * cc + b, buf, sem)
            return carry

        lax.fori_loop(0, n_chunk // 2, body, 0)
        wait_put(rows0, p0)
        wait_put(rows1, p1)

    return lookup


def _tc_block(z_ref, tab_ref, out_ref, *, n_tab):
    idx = z_ref[...]
    idx = jnp.where(idx > -1, idx + 1, n_tab - 1)
    onehot = (idx[:, None] == lax.iota(jnp.int32, n_tab)[None, :])
    out_ref[...] = jax.lax.dot_general(
        onehot.astype(tab_ref.dtype), tab_ref[...],
        (((1,), (0,)), ((), ())),
        preferred_element_type=jnp.float32)


def _make_tc_lookup(n_rows, n_tab, d, dtype):
    grid = (n_rows // BLK,)
    return pl.pallas_call(
        functools.partial(_tc_block, n_tab=n_tab),
        grid=grid,
        in_specs=[
            pl.BlockSpec((BLK,), lambda i: (i,)),
            pl.BlockSpec((n_tab, d), lambda i: (0, 0)),
        ],
        out_specs=pl.BlockSpec((BLK, d), lambda i: (i, 0)),
        out_shape=jax.ShapeDtypeStruct((n_rows, d), dtype),
    )


def kernel(z, r, table):
    b, a = z.shape
    n_tab, d = table.shape
    n_rows = b * a
    zf = z.reshape(-1).astype(jnp.int32)
    tpad = jnp.concatenate([table, jnp.zeros((1, d), table.dtype)], axis=0)

    n_sc = (n_rows * SC_FRAC_NUM // 100) // SC_ALIGN * SC_ALIGN
    if n_sc == 0 or (n_rows - n_sc) % BLK != 0:
        n_sc = n_rows

    emb_sc = _make_sc_lookup(n_sc, n_tab + 1, d, table.dtype)(
        zf[:n_sc], tpad.reshape(-1)).reshape(n_sc, d)
    if n_sc == n_rows:
        emb = emb_sc
    else:
        emb_tc = _make_tc_lookup(n_rows - n_sc, n_tab + 1, d, table.dtype)(
            zf[n_sc:], tpad)
        emb = jnp.concatenate([emb_sc, emb_tc], axis=0)
    return (z, r, emb.reshape(b, a, d))
